# baseline (speedup 1.0000x reference)
import functools

import jax
import jax.numpy as jnp
from jax.experimental import pallas as pl
from jax.experimental.pallas import tpu as pltpu

_EPS = 1e-6
_GRID = 2


def _angle_loss_kernel(n: int, feat_ref, true_ref, mask_ref, out_ref,
                       a_acc, b_acc):
    step = pl.program_id(0)
    t_count = float(n * (n - 1) * (n - 2))
    dn = (((2,), (2,)), ((0,), (0,)))

    f = feat_ref[:]
    u = true_ref[:]
    ga = jax.lax.dot_general(f, f, dn, preferred_element_type=jnp.float32)
    gb = jax.lax.dot_general(u, u, dn, preferred_element_type=jnp.float32)

    @pl.when(step == 0)
    def _init():
        a_acc[:] = ga
        b_acc[:] = gb

    @pl.when(step > 0)
    def _accum():
        a_acc[:] += ga
        b_acc[:] += gb

    @pl.when(step == _GRID - 1)
    def _finalize():
        ii = jax.lax.broadcasted_iota(jnp.int32, (n, n, n), 0)
        jj = jax.lax.broadcasted_iota(jnp.int32, (n, n, n), 1)
        kk = jax.lax.broadcasted_iota(jnp.int32, (n, n, n), 2)
        diag_m = (jj == kk).astype(jnp.float32)
        valid = (jj != ii) & (kk != ii) & (jj != kk)
        vf = valid.astype(jnp.float32)

        def _normalize(g):
            diag = jnp.sum(g * diag_m, axis=-1)
            inv = jax.lax.rsqrt(jnp.maximum(diag, _EPS * _EPS))
            return g * inv[:, :, None] * inv[:, None, :]

        a = _normalize(a_acc[:])
        b = _normalize(b_acc[:])

        bv = b * vf
        s1 = jnp.sum(bv)
        s2 = jnp.sum(bv * b)
        ae = a + _EPS
        q = t_count * ae * ae - 2.0 * ae * s1 + s2
        total = jnp.sum(jnp.sqrt(jnp.maximum(q, 0.0)) * vf)

        m = mask_ref[:].astype(jnp.float32)
        mi = jax.lax.broadcasted_iota(jnp.int32, (n, n), 0)
        mj = jax.lax.broadcasted_iota(jnp.int32, (n, n), 1)
        gate = jnp.min(jnp.where(mi == mj, 1.0, m))

        out_ref[0, 0] = total / t_count * gate * 0.5


def kernel(feat_angle_dist_matrix, positive_masks, true_angle_dist_matrix):
    n = positive_masks.shape[0]
    d = feat_angle_dist_matrix.shape[-1]
    ch = d // _GRID
    out = pl.pallas_call(
        functools.partial(_angle_loss_kernel, n),
        grid=(_GRID,),
        in_specs=[
            pl.BlockSpec((n, n, ch), lambda s: (0, 0, s)),
            pl.BlockSpec((n, n, ch), lambda s: (0, 0, s)),
            pl.BlockSpec((n, n), lambda s: (0, 0)),
        ],
        out_shape=jax.ShapeDtypeStruct((1, 1), jnp.float32),
        out_specs=pl.BlockSpec(memory_space=pltpu.SMEM),
        scratch_shapes=[
            pltpu.VMEM((n, n, n), jnp.float32),
            pltpu.VMEM((n, n, n), jnp.float32),
        ],
    )(feat_angle_dist_matrix, true_angle_dist_matrix, positive_masks)
    return out.reshape(())


# speedup vs baseline: 1.0500x; 1.0500x over previous
import functools

import jax
import jax.numpy as jnp
from jax.experimental import pallas as pl
from jax.experimental.pallas import tpu as pltpu

_EPS = 1e-6


def _angle_loss_kernel(n: int, feat_ref, true_ref, mask_ref, out_ref):
    t_count = float(n * (n - 1) * (n - 2))

    ii = jax.lax.broadcasted_iota(jnp.int32, (n, n, n), 0)
    jj = jax.lax.broadcasted_iota(jnp.int32, (n, n, n), 1)
    kk = jax.lax.broadcasted_iota(jnp.int32, (n, n, n), 2)
    diag_m = (jj == kk).astype(jnp.float32)
    valid = (jj != ii) & (kk != ii) & (jj != kk)
    vf = valid.astype(jnp.float32)

    dn = (((2,), (2,)), ((0,), (0,)))

    def _norm_gram(x):
        g = jax.lax.dot_general(x, x, dn, preferred_element_type=jnp.float32)
        diag = jnp.sum(g * diag_m, axis=-1)
        inv = jax.lax.rsqrt(jnp.maximum(diag, _EPS * _EPS))
        return g * inv[:, :, None] * inv[:, None, :]

    a = _norm_gram(feat_ref[:])
    b = _norm_gram(true_ref[:])

    bv = b * vf
    s1 = jnp.sum(bv)
    s2 = jnp.sum(bv * b)
    ae = a + _EPS
    q = t_count * ae * ae - 2.0 * ae * s1 + s2
    total = jnp.sum(jnp.sqrt(jnp.maximum(q, 0.0)) * vf)

    m = mask_ref[:].astype(jnp.float32)
    mi = jax.lax.broadcasted_iota(jnp.int32, (n, n), 0)
    mj = jax.lax.broadcasted_iota(jnp.int32, (n, n), 1)
    gate = jnp.min(jnp.where(mi == mj, 1.0, m))

    out_ref[0, 0] = total / t_count * gate * 0.5


def kernel(feat_angle_dist_matrix, positive_masks, true_angle_dist_matrix):
    n = positive_masks.shape[0]
    out = pl.pallas_call(
        functools.partial(_angle_loss_kernel, n),
        out_shape=jax.ShapeDtypeStruct((1, 1), jnp.float32),
        out_specs=pl.BlockSpec(memory_space=pltpu.SMEM),
    )(feat_angle_dist_matrix, true_angle_dist_matrix, positive_masks)
    return out.reshape(())
